# Initial kernel scaffold; baseline (speedup 1.0000x reference)
import jax
import jax.numpy as jnp
from jax.experimental import pallas as pl


def kernel(x, edge_index, W1, att_src1, att_dst1, b1, W2, att_src2, att_dst2, b2):
    raise NotImplementedError("write your pallas kernel here")


# speedup vs baseline: 18.0001x; 18.0001x over previous
import functools

import jax
import jax.numpy as jnp
from jax import lax
from jax.experimental import pallas as pl
from jax.experimental.pallas import tpu as pltpu
from jax.experimental.pallas import tpu_sc as plsc

N = 10000
E = 320000
D = 128
NPAD = 10240
NC, NS, L = 2, 16, 16
NW = NC * NS
EPT = E // NW
CW = 80
NCH = EPT // CW
ROWS = E // CW
SLICE = NPAD // NS


def _full16(v):
    return jnp.full((L,), v, dtype=jnp.int32)


def _lrelu(a):
    return jnp.where(a >= 0.0, a, a * jnp.float32(0.2))


@functools.lru_cache(maxsize=None)
def _sc_kernels():
    mesh = plsc.VectorSubcoreMesh(
        core_axis_name="c", subcore_axis_name="s",
        num_cores=NC, num_subcores=NS)

    @functools.partial(
        pl.kernel,
        out_type=(
            jax.ShapeDtypeStruct((ROWS, 1, CW), jnp.float32),
            jax.ShapeDtypeStruct((NC, NPAD), jnp.float32),
            jax.ShapeDtypeStruct((NPAD,), jnp.float32),
        ),
        mesh=mesh,
        compiler_params=pltpu.CompilerParams(needs_layout_passes=False),
        scratch_types=[
            pltpu.VMEM((NPAD,), jnp.float32),
            pltpu.VMEM((NPAD,), jnp.float32),
            pltpu.VMEM((NCH, 1, CW), jnp.int32),
            pltpu.VMEM((NCH, 1, CW), jnp.int32),
            pltpu.VMEM((NCH, 1, CW), jnp.float32),
            pltpu.VMEM((SLICE,), jnp.float32),
            pltpu.VMEM((SLICE,), jnp.float32),
            pltpu.VMEM_SHARED((NPAD,), jnp.float32),
        ],
    )
    def sc_edge(asrc_hbm, adst_hbm, src_hbm, dst_hbm,
                ex_out, den_out, es_out,
                asrc_v, adst_v, src_v, dst_v, ex_v, es_v, init_v, den_sp):
        cid = lax.axis_index("c")
        sid = lax.axis_index("s")
        wid = cid * NS + sid

        pltpu.sync_copy(asrc_hbm, asrc_v)
        pltpu.sync_copy(adst_hbm, adst_v)
        pltpu.sync_copy(src_hbm.at[pl.ds(wid * NCH, NCH)], src_v)
        pltpu.sync_copy(dst_hbm.at[pl.ds(wid * NCH, NCH)], dst_v)

        def edge_body(ci, _):
            for j in range(CW // L):
                sl = pl.ds(j * L, L)
                sv = src_v[ci, 0, sl]
                dv = dst_v[ci, 0, sl]
                a = (plsc.load_gather(asrc_v, [sv])
                     + plsc.load_gather(adst_v, [dv]))
                ex_v[ci, 0, sl] = jnp.exp(_lrelu(a))
            return _

        lax.fori_loop(0, NCH, edge_body, None)
        pltpu.sync_copy(ex_v, ex_out.at[pl.ds(wid * NCH, NCH)])

        base = sid * SLICE
        iszero = cid == 0
        for r in range(SLICE // L):
            sl = pl.ds(base + r * L, L)
            es = jnp.exp(_lrelu(asrc_v[sl] + adst_v[sl]))
            es_v[pl.ds(r * L, L)] = es
            init_v[pl.ds(r * L, L)] = jnp.where(iszero, es, jnp.float32(0.0))
        pltpu.sync_copy(init_v, den_sp.at[pl.ds(base, SLICE)])

        @pl.when(cid == 0)
        def _():
            pltpu.sync_copy(es_v, es_out.at[pl.ds(base, SLICE)])

        plsc.subcore_barrier()

        def scat_body(ci, _):
            pltpu.sync_copy(ex_v.at[ci, 0], den_sp.at[dst_v.at[ci, 0]], add=True)
            return _

        lax.fori_loop(0, NCH, scat_body, None)

        plsc.subcore_barrier()

        pltpu.sync_copy(den_sp.at[pl.ds(base, SLICE)],
                        den_out.at[cid, pl.ds(base, SLICE)])

    @functools.partial(
        pl.kernel,
        out_type=jax.ShapeDtypeStruct((NC, NPAD, D), jnp.float32),
        mesh=mesh,
        compiler_params=pltpu.CompilerParams(needs_layout_passes=False),
        scratch_types=[
            pltpu.VMEM((NPAD,), jnp.float32),
            pltpu.VMEM((SLICE,), jnp.float32),
            pltpu.VMEM((1, CW), jnp.int32),
            pltpu.VMEM((1, CW), jnp.int32),
            pltpu.VMEM((1, CW), jnp.float32),
            pltpu.VMEM((CW + L,), jnp.float32),
            pltpu.VMEM((CW, D), jnp.float32),
            pltpu.VMEM_SHARED((NPAD, D), jnp.float32),
        ],
    )
    def sc_agg(h_hbm, src_hbm, dst_hbm, ex_hbm, den_hbm, es_hbm,
               p_out,
               den_v, es_v, srcc_v, dstc_v, exc_v, coef_v, rows_v, out_sp):
        cid = lax.axis_index("c")
        sid = lax.axis_index("s")
        wid = cid * NS + sid
        base = sid * SLICE

        pltpu.sync_copy(den_hbm, den_v)
        pltpu.sync_copy(es_hbm.at[pl.ds(base, SLICE)], es_v)

        @pl.when(cid == 0)
        def _():
            def init_body(b, _):
                row0 = base + b * L
                pltpu.sync_copy(h_hbm.at[pl.ds(row0, L)],
                                rows_v.at[pl.ds(0, L)])
                coef_v[pl.ds(L, L)] = (es_v[pl.ds(b * L, L)]
                                       / den_v[pl.ds(row0, L)])
                for rr in range(L):
                    bc = plsc.load_gather(coef_v, [_full16(L + rr)])
                    for k in range(D // L):
                        sl = pl.ds(k * L, L)
                        rows_v[rr, sl] = rows_v[rr, sl] * bc
                pltpu.sync_copy(rows_v.at[pl.ds(0, L)],
                                out_sp.at[pl.ds(row0, L)])
                return _

            lax.fori_loop(0, SLICE // L, init_body, None)

        @pl.when(cid == 1)
        def _():
            for rr in range(L):
                for k in range(D // L):
                    rows_v[rr, pl.ds(k * L, L)] = jnp.zeros((L,), jnp.float32)

            def zero_body(b, _):
                pltpu.sync_copy(rows_v.at[pl.ds(0, L)],
                                out_sp.at[pl.ds(base + b * L, L)])
                return _

            lax.fori_loop(0, SLICE // L, zero_body, None)

        plsc.subcore_barrier()

        def agg_body(ci, _):
            row = wid * NCH + ci
            pltpu.sync_copy(src_hbm.at[row], srcc_v)
            pltpu.sync_copy(dst_hbm.at[row], dstc_v)
            pltpu.sync_copy(ex_hbm.at[row], exc_v)
            for j in range(CW // L):
                sl = pl.ds(j * L, L)
                dv = dstc_v[0, sl]
                den16 = plsc.load_gather(den_v, [dv])
                coef_v[pl.ds(L + j * L, L)] = exc_v[0, sl] / den16
            pltpu.sync_copy(h_hbm.at[srcc_v.at[0]], rows_v)
            for j in range(CW):
                bc = plsc.load_gather(coef_v, [_full16(L + j)])
                for k in range(D // L):
                    sl = pl.ds(k * L, L)
                    rows_v[j, sl] = rows_v[j, sl] * bc
            pltpu.sync_copy(rows_v, out_sp.at[dstc_v.at[0]], add=True)
            return _

        lax.fori_loop(0, NCH, agg_body, None)

        plsc.subcore_barrier()

        pltpu.sync_copy(out_sp.at[pl.ds(base, SLICE)],
                        p_out.at[cid, pl.ds(base, SLICE)])

    return sc_edge, sc_agg


_BLK = 1024


def _tc_pre_body(x_ref, w_ref, att_ref, h_ref, a_ref):
    h = jnp.dot(x_ref[...], w_ref[...], preferred_element_type=jnp.float32)
    h_ref[...] = h
    a_ref[...] = jnp.dot(h, att_ref[...], preferred_element_type=jnp.float32)


def _tc_pre(x, w, att):
    return pl.pallas_call(
        _tc_pre_body,
        grid=(NPAD // _BLK,),
        in_specs=[
            pl.BlockSpec((_BLK, D), lambda i: (i, 0)),
            pl.BlockSpec((D, D), lambda i: (0, 0)),
            pl.BlockSpec((D, D), lambda i: (0, 0)),
        ],
        out_specs=[
            pl.BlockSpec((_BLK, D), lambda i: (i, 0)),
            pl.BlockSpec((_BLK, D), lambda i: (i, 0)),
        ],
        out_shape=[
            jax.ShapeDtypeStruct((NPAD, D), jnp.float32),
            jax.ShapeDtypeStruct((NPAD, D), jnp.float32),
        ],
    )(x, w, att)


def _tc_den_body(dp_ref, den_ref):
    den_ref[...] = dp_ref[0, :] + dp_ref[1, :] + jnp.float32(1e-16)


def _tc_den(dp):
    return pl.pallas_call(
        _tc_den_body,
        in_specs=[pl.BlockSpec((NC, NPAD), lambda: (0, 0))],
        out_specs=pl.BlockSpec((NPAD,), lambda: (0,)),
        out_shape=jax.ShapeDtypeStruct((NPAD,), jnp.float32),
    )(dp)


def _tc_mid_body(p0_ref, p1_ref, b_ref, w_ref, att_ref, h_ref, a_ref):
    out1 = p0_ref[...] + p1_ref[...] + b_ref[...]
    hm = jnp.where(out1 > 0.0, out1, jnp.exp(out1) - 1.0)
    h = jnp.dot(hm, w_ref[...], preferred_element_type=jnp.float32)
    h_ref[...] = h
    a_ref[...] = jnp.dot(h, att_ref[...], preferred_element_type=jnp.float32)


def _tc_mid(p0, p1, b, w, att):
    return pl.pallas_call(
        _tc_mid_body,
        grid=(NPAD // _BLK,),
        in_specs=[
            pl.BlockSpec((_BLK, D), lambda i: (i, 0)),
            pl.BlockSpec((_BLK, D), lambda i: (i, 0)),
            pl.BlockSpec((1, D), lambda i: (0, 0)),
            pl.BlockSpec((D, D), lambda i: (0, 0)),
            pl.BlockSpec((D, D), lambda i: (0, 0)),
        ],
        out_specs=[
            pl.BlockSpec((_BLK, D), lambda i: (i, 0)),
            pl.BlockSpec((_BLK, D), lambda i: (i, 0)),
        ],
        out_shape=[
            jax.ShapeDtypeStruct((NPAD, D), jnp.float32),
            jax.ShapeDtypeStruct((NPAD, D), jnp.float32),
        ],
    )(p0, p1, b, w, att)


def _tc_final_body(p0_ref, p1_ref, b_ref, o_ref):
    o_ref[...] = p0_ref[...] + p1_ref[...] + b_ref[...]


def _tc_final(p0, p1, b):
    return pl.pallas_call(
        _tc_final_body,
        grid=(NPAD // _BLK,),
        in_specs=[
            pl.BlockSpec((_BLK, D), lambda i: (i, 0)),
            pl.BlockSpec((_BLK, D), lambda i: (i, 0)),
            pl.BlockSpec((1, D), lambda i: (0, 0)),
        ],
        out_specs=pl.BlockSpec((_BLK, D), lambda i: (i, 0)),
        out_shape=jax.ShapeDtypeStruct((NPAD, D), jnp.float32),
    )(p0, p1, b)


def _att_mat(att_src, att_dst):
    vs = att_src.reshape(D)
    vd = att_dst.reshape(D)
    return jnp.concatenate(
        [vs[:, None], vd[:, None], jnp.zeros((D, D - 2), jnp.float32)], axis=1)


def _gat_layer(h, asrc, adst, src3d, dst3d):
    sc_edge, sc_agg = _sc_kernels()
    ex, dp, es = sc_edge(asrc, adst, src3d, dst3d)
    den = _tc_den(dp)
    parts = sc_agg(h, src3d, dst3d, ex, den, es)
    return parts[0], parts[1]


def _impl(x, edge_index, W1, att_src1, att_dst1, b1, W2, att_src2, att_dst2, b2):
    x_pad = jnp.pad(x, ((0, NPAD - N), (0, 0)))
    src3d = edge_index[0].reshape(ROWS, 1, CW)
    dst3d = edge_index[1].reshape(ROWS, 1, CW)

    h1, a1 = _tc_pre(x_pad, W1, _att_mat(att_src1, att_dst1))
    p0, p1 = _gat_layer(h1, a1[:, 0], a1[:, 1], src3d, dst3d)

    h2, a2 = _tc_mid(p0, p1, b1.reshape(1, D), W2, _att_mat(att_src2, att_dst2))
    q0, q1 = _gat_layer(h2, a2[:, 0], a2[:, 1], src3d, dst3d)

    out = _tc_final(q0, q1, b2.reshape(1, D))
    return out[:N]


kernel = jax.jit(_impl)


# speedup vs baseline: 20.7557x; 1.1531x over previous
import functools

import jax
import jax.numpy as jnp
from jax import lax
from jax.experimental import pallas as pl
from jax.experimental.pallas import tpu as pltpu
from jax.experimental.pallas import tpu_sc as plsc

N = 10000
E = 320000
D = 128
NPAD = 10240
NC, NS, L = 2, 16, 16
NW = NC * NS
EPT = E // NW
CW = 80
NCH = EPT // CW
ROWS = E // CW
CWA = 40
NCHA = EPT // CWA
ROWSA = E // CWA
NBUF = 4
SLICE = NPAD // NS


def _full16(v):
    return jnp.full((L,), v, dtype=jnp.int32)


def _lrelu(a):
    return jnp.where(a >= 0.0, a, a * jnp.float32(0.2))


@functools.lru_cache(maxsize=None)
def _sc_kernels():
    mesh = plsc.VectorSubcoreMesh(
        core_axis_name="c", subcore_axis_name="s",
        num_cores=NC, num_subcores=NS)

    @functools.partial(
        pl.kernel,
        out_type=(
            jax.ShapeDtypeStruct((ROWS, 1, CW), jnp.float32),
            jax.ShapeDtypeStruct((NC, NPAD), jnp.float32),
            jax.ShapeDtypeStruct((NPAD,), jnp.float32),
        ),
        mesh=mesh,
        compiler_params=pltpu.CompilerParams(needs_layout_passes=False),
        scratch_types=[
            pltpu.VMEM((NPAD,), jnp.float32),
            pltpu.VMEM((NPAD,), jnp.float32),
            pltpu.VMEM((NCH, 1, CW), jnp.int32),
            pltpu.VMEM((NCH, 1, CW), jnp.int32),
            pltpu.VMEM((NCH, 1, CW), jnp.float32),
            pltpu.VMEM((SLICE,), jnp.float32),
            pltpu.VMEM((SLICE,), jnp.float32),
            pltpu.VMEM_SHARED((NPAD,), jnp.float32),
        ],
    )
    def sc_edge(asrc_hbm, adst_hbm, src_hbm, dst_hbm,
                ex_out, den_out, es_out,
                asrc_v, adst_v, src_v, dst_v, ex_v, es_v, init_v, den_sp):
        cid = lax.axis_index("c")
        sid = lax.axis_index("s")
        wid = cid * NS + sid

        pltpu.sync_copy(asrc_hbm, asrc_v)
        pltpu.sync_copy(adst_hbm, adst_v)
        pltpu.sync_copy(src_hbm.at[pl.ds(wid * NCH, NCH)], src_v)
        pltpu.sync_copy(dst_hbm.at[pl.ds(wid * NCH, NCH)], dst_v)

        def edge_body(ci, _):
            for j in range(CW // L):
                sl = pl.ds(j * L, L)
                sv = src_v[ci, 0, sl]
                dv = dst_v[ci, 0, sl]
                a = (plsc.load_gather(asrc_v, [sv])
                     + plsc.load_gather(adst_v, [dv]))
                ex_v[ci, 0, sl] = jnp.exp(_lrelu(a))
            return _

        lax.fori_loop(0, NCH, edge_body, None)
        pltpu.sync_copy(ex_v, ex_out.at[pl.ds(wid * NCH, NCH)])

        base = sid * SLICE
        iszero = cid == 0
        for r in range(SLICE // L):
            sl = pl.ds(base + r * L, L)
            es = jnp.exp(_lrelu(asrc_v[sl] + adst_v[sl]))
            es_v[pl.ds(r * L, L)] = es
            init_v[pl.ds(r * L, L)] = jnp.where(iszero, es, jnp.float32(0.0))
        pltpu.sync_copy(init_v, den_sp.at[pl.ds(base, SLICE)])

        @pl.when(cid == 0)
        def _():
            pltpu.sync_copy(es_v, es_out.at[pl.ds(base, SLICE)])

        plsc.subcore_barrier()

        def scat_body(ci, _):
            pltpu.sync_copy(ex_v.at[ci, 0], den_sp.at[dst_v.at[ci, 0]], add=True)
            return _

        lax.fori_loop(0, NCH, scat_body, None)

        plsc.subcore_barrier()

        pltpu.sync_copy(den_sp.at[pl.ds(base, SLICE)],
                        den_out.at[cid, pl.ds(base, SLICE)])

    @functools.partial(
        pl.kernel,
        out_type=(
            jax.ShapeDtypeStruct((ROWS, 1, CW), jnp.float32),
            jax.ShapeDtypeStruct((NPAD,), jnp.float32),
        ),
        mesh=mesh,
        compiler_params=pltpu.CompilerParams(needs_layout_passes=False),
        scratch_types=[
            pltpu.VMEM((NPAD,), jnp.float32),
            pltpu.VMEM((NPAD,), jnp.float32),
            pltpu.VMEM((NCH, 1, CW), jnp.int32),
            pltpu.VMEM((NCH, 1, CW), jnp.float32),
            pltpu.VMEM((SLICE,), jnp.float32),
        ],
    )
    def sc_norm(dp_hbm, dst_hbm, ex_hbm, es_hbm,
                coef_out, selfc_out,
                den_v, tmp_v, dst_v, ex_v, es_v):
        cid = lax.axis_index("c")
        sid = lax.axis_index("s")
        wid = cid * NS + sid
        base = sid * SLICE

        pltpu.sync_copy(dp_hbm.at[0], den_v)
        pltpu.sync_copy(dp_hbm.at[1], tmp_v)

        def den_body(r, _):
            sl = pl.ds(r * L, L)
            den_v[sl] = den_v[sl] + tmp_v[sl] + jnp.float32(1e-16)
            return _

        lax.fori_loop(0, NPAD // L, den_body, None)

        pltpu.sync_copy(dst_hbm.at[pl.ds(wid * NCH, NCH)], dst_v)
        pltpu.sync_copy(ex_hbm.at[pl.ds(wid * NCH, NCH)], ex_v)

        def coef_body(ci, _):
            for j in range(CW // L):
                sl = pl.ds(j * L, L)
                dv = dst_v[ci, 0, sl]
                den16 = plsc.load_gather(den_v, [dv])
                ex_v[ci, 0, sl] = ex_v[ci, 0, sl] / den16
            return _

        lax.fori_loop(0, NCH, coef_body, None)
        pltpu.sync_copy(ex_v, coef_out.at[pl.ds(wid * NCH, NCH)])

        pltpu.sync_copy(es_hbm.at[pl.ds(base, SLICE)], es_v)
        for r in range(SLICE // L):
            sl = pl.ds(r * L, L)
            es_v[sl] = es_v[sl] / den_v[pl.ds(base + r * L, L)]
        pltpu.sync_copy(es_v, selfc_out.at[pl.ds(base, SLICE)])

    _g16 = CWA // 8

    @functools.partial(
        pl.kernel,
        out_type=jax.ShapeDtypeStruct((NC, NPAD, D), jnp.float32),
        mesh=mesh,
        compiler_params=pltpu.CompilerParams(needs_layout_passes=False),
        scratch_types=(
            [pltpu.VMEM((SLICE,), jnp.float32)]
            + [pltpu.VMEM((2, CWA), jnp.int32)] * NBUF
            + [pltpu.VMEM((1, CWA), jnp.float32)] * NBUF
            + [pltpu.VMEM((CWA, D), jnp.float32)] * NBUF
            + [pltpu.SemaphoreType.DMA] * (2 * NBUF)
            + [pltpu.VMEM_SHARED((NPAD, D), jnp.float32)]
        ),
    )
    def sc_agg(h_hbm, sd_hbm, coef_hbm, selfc_hbm,
               p_out,
               selfc_v, sd0, sd1, sd2, sd3, cf0, cf1, cf2, cf3,
               rw0, rw1, rw2, rw3, gs0, gs1, gs2, gs3, ss0, ss1, ss2, ss3,
               out_sp):
        cid = lax.axis_index("c")
        sid = lax.axis_index("s")
        wid = cid * NS + sid
        base = sid * SLICE
        sd = [sd0, sd1, sd2, sd3]
        cf = [cf0, cf1, cf2, cf3]
        rw = [rw0, rw1, rw2, rw3]
        gs = [gs0, gs1, gs2, gs3]
        ss = [ss0, ss1, ss2, ss3]

        pltpu.sync_copy(selfc_hbm.at[pl.ds(base, SLICE)], selfc_v)

        @pl.when(cid == 0)
        def _():
            def init_body(b, _):
                row0 = base + b * L
                pltpu.sync_copy(h_hbm.at[pl.ds(row0, L)],
                                rw0.at[pl.ds(0, L)])
                for rr in range(L):
                    bc = plsc.load_gather(selfc_v, [_full16(b * L + rr)])
                    for k in range(D // L):
                        sl = pl.ds(k * L, L)
                        rw0[rr, sl] = rw0[rr, sl] * bc
                pltpu.sync_copy(rw0.at[pl.ds(0, L)],
                                out_sp.at[pl.ds(row0, L)])
                return _

            lax.fori_loop(0, SLICE // L, init_body, None)

        @pl.when(cid == 1)
        def _():
            for rr in range(L):
                for k in range(D // L):
                    rw0[rr, pl.ds(k * L, L)] = jnp.zeros((L,), jnp.float32)

            def zero_body(b, _):
                pltpu.sync_copy(rw0.at[pl.ds(0, L)],
                                out_sp.at[pl.ds(base + b * L, L)])
                return _

            lax.fori_loop(0, SLICE // L, zero_body, None)

        plsc.subcore_barrier()

        cbase = wid * NCHA

        def load_smalls(b, c):
            pltpu.sync_copy(sd_hbm.at[cbase + c], sd[b])
            pltpu.sync_copy(coef_hbm.at[cbase + c], cf[b])

        def start_gather(b):
            pltpu.async_copy(h_hbm.at[sd[b].at[0]], rw[b], gs[b])

        def wait_gather(b):
            pltpu.make_async_copy(h_hbm.at[sd[b].at[0]], rw[b], gs[b]).wait()

        def start_scatter(b):
            pltpu.async_copy(rw[b], out_sp.at[sd[b].at[1]], ss[b], add=True)

        def wait_scatter(b):
            pltpu.make_async_copy(rw[b], out_sp.at[sd[b].at[1]],
                                  ss[b]).wait()

        def scale(b):
            def grp(g, _):
                for rr in range(8):
                    bc = plsc.load_gather(
                        cf[b], [_full16(0), _full16(g * 8 + rr)])
                    for k in range(D // L):
                        sl = pl.ds(k * L, L)
                        rw[b][g * 8 + rr, sl] = rw[b][g * 8 + rr, sl] * bc
                return _

            lax.fori_loop(0, _g16, grp, None)

        load_smalls(0, 0)
        start_gather(0)
        load_smalls(1, 1)
        start_gather(1)

        def pipe_body(i, _):
            c0 = i * NBUF
            for u in range(NBUF):
                b = u
                zb = (u + 2) % NBUF
                c = c0 + u
                wait_gather(b)
                scale(b)
                start_scatter(b)
                if u < 2:
                    @pl.when(i > 0)
                    def _(zb=zb):
                        wait_scatter(zb)
                else:
                    wait_scatter(zb)
                load_smalls(zb, c + 2)
                start_gather(zb)
            return _

        lax.fori_loop(0, (NCHA - 2) // NBUF, pipe_body, None)

        for u in range(2):
            b = (NCHA - 2 + u) % NBUF
            wait_gather(b)
            scale(b)
            start_scatter(b)
        for b in range(NBUF):
            wait_scatter(b)

        plsc.subcore_barrier()

        pltpu.sync_copy(out_sp.at[pl.ds(base, SLICE)],
                        p_out.at[cid, pl.ds(base, SLICE)])

    return sc_edge, sc_norm, sc_agg


_BLK = 1024


def _tc_pre_body(x_ref, w_ref, att_ref, h_ref, a_ref):
    h = jnp.dot(x_ref[...], w_ref[...], preferred_element_type=jnp.float32)
    h_ref[...] = h
    a_ref[...] = jnp.dot(h, att_ref[...], preferred_element_type=jnp.float32)


def _tc_pre(x, w, att):
    return pl.pallas_call(
        _tc_pre_body,
        grid=(NPAD // _BLK,),
        in_specs=[
            pl.BlockSpec((_BLK, D), lambda i: (i, 0)),
            pl.BlockSpec((D, D), lambda i: (0, 0)),
            pl.BlockSpec((D, D), lambda i: (0, 0)),
        ],
        out_specs=[
            pl.BlockSpec((_BLK, D), lambda i: (i, 0)),
            pl.BlockSpec((_BLK, D), lambda i: (i, 0)),
        ],
        out_shape=[
            jax.ShapeDtypeStruct((NPAD, D), jnp.float32),
            jax.ShapeDtypeStruct((NPAD, D), jnp.float32),
        ],
    )(x, w, att)


def _tc_den_body(dp_ref, den_ref):
    den_ref[...] = dp_ref[0, :] + dp_ref[1, :] + jnp.float32(1e-16)


def _tc_den(dp):
    return pl.pallas_call(
        _tc_den_body,
        in_specs=[pl.BlockSpec((NC, NPAD), lambda: (0, 0))],
        out_specs=pl.BlockSpec((NPAD,), lambda: (0,)),
        out_shape=jax.ShapeDtypeStruct((NPAD,), jnp.float32),
    )(dp)


def _tc_mid_body(p0_ref, p1_ref, b_ref, w_ref, att_ref, h_ref, a_ref):
    out1 = p0_ref[...] + p1_ref[...] + b_ref[...]
    hm = jnp.where(out1 > 0.0, out1, jnp.exp(out1) - 1.0)
    h = jnp.dot(hm, w_ref[...], preferred_element_type=jnp.float32)
    h_ref[...] = h
    a_ref[...] = jnp.dot(h, att_ref[...], preferred_element_type=jnp.float32)


def _tc_mid(p0, p1, b, w, att):
    return pl.pallas_call(
        _tc_mid_body,
        grid=(NPAD // _BLK,),
        in_specs=[
            pl.BlockSpec((_BLK, D), lambda i: (i, 0)),
            pl.BlockSpec((_BLK, D), lambda i: (i, 0)),
            pl.BlockSpec((1, D), lambda i: (0, 0)),
            pl.BlockSpec((D, D), lambda i: (0, 0)),
            pl.BlockSpec((D, D), lambda i: (0, 0)),
        ],
        out_specs=[
            pl.BlockSpec((_BLK, D), lambda i: (i, 0)),
            pl.BlockSpec((_BLK, D), lambda i: (i, 0)),
        ],
        out_shape=[
            jax.ShapeDtypeStruct((NPAD, D), jnp.float32),
            jax.ShapeDtypeStruct((NPAD, D), jnp.float32),
        ],
    )(p0, p1, b, w, att)


def _tc_final_body(p0_ref, p1_ref, b_ref, o_ref):
    o_ref[...] = p0_ref[...] + p1_ref[...] + b_ref[...]


def _tc_final(p0, p1, b):
    return pl.pallas_call(
        _tc_final_body,
        grid=(NPAD // _BLK,),
        in_specs=[
            pl.BlockSpec((_BLK, D), lambda i: (i, 0)),
            pl.BlockSpec((_BLK, D), lambda i: (i, 0)),
            pl.BlockSpec((1, D), lambda i: (0, 0)),
        ],
        out_specs=pl.BlockSpec((_BLK, D), lambda i: (i, 0)),
        out_shape=jax.ShapeDtypeStruct((NPAD, D), jnp.float32),
    )(p0, p1, b)


def _att_mat(att_src, att_dst):
    vs = att_src.reshape(D)
    vd = att_dst.reshape(D)
    return jnp.concatenate(
        [vs[:, None], vd[:, None], jnp.zeros((D, D - 2), jnp.float32)], axis=1)


def _gat_layer(h, asrc, adst, src3d, dst3d, sd3):
    sc_edge, sc_norm, sc_agg = _sc_kernels()
    ex, dp, es = sc_edge(asrc, adst, src3d, dst3d)
    coef, selfc = sc_norm(dp, dst3d, ex, es)
    parts = sc_agg(h, sd3, coef.reshape(ROWSA, 1, CWA), selfc)
    return parts[0], parts[1]


def _impl(x, edge_index, W1, att_src1, att_dst1, b1, W2, att_src2, att_dst2, b2):
    x_pad = jnp.pad(x, ((0, NPAD - N), (0, 0)))
    src3d = edge_index[0].reshape(ROWS, 1, CW)
    dst3d = edge_index[1].reshape(ROWS, 1, CW)
    sd3 = jnp.stack([edge_index[0].reshape(ROWSA, CWA),
                     edge_index[1].reshape(ROWSA, CWA)], axis=1)

    h1, a1 = _tc_pre(x_pad, W1, _att_mat(att_src1, att_dst1))
    p0, p1 = _gat_layer(h1, a1[:, 0], a1[:, 1], src3d, dst3d, sd3)

    h2, a2 = _tc_mid(p0, p1, b1.reshape(1, D), W2, _att_mat(att_src2, att_dst2))
    q0, q1 = _gat_layer(h2, a2[:, 0], a2[:, 1], src3d, dst3d, sd3)

    out = _tc_final(q0, q1, b2.reshape(1, D))
    return out[:N]


kernel = jax.jit(_impl)
